# baseline (speedup 1.0000x reference)
import jax
import jax.numpy as jnp
from jax.experimental import pallas as pl
from jax.experimental.pallas import tpu as pltpu

_B, _N, _L = 4, 4096, 1024
_BLK = 2048
_CH = 256
_TILES_PER_BATCH = _N // _BLK


def _scan_body(x_ref, o_ref, carry_ref):
    i = pl.program_id(0)

    @pl.when(i % _TILES_PER_BATCH == 0)
    def _():
        carry_ref[...] = jnp.zeros_like(carry_ref)

    rows = jax.lax.broadcasted_iota(jnp.int32, (_CH, _CH), 0)
    cols = jax.lax.broadcasted_iota(jnp.int32, (_CH, _CH), 1)
    tri = (cols < rows).astype(jnp.float32)

    tot = carry_ref[...]
    for c in range(_BLK // _CH):
        xc = x_ref[pl.ds(c * _CH, _CH), :]
        excl = jnp.dot(tri, xc, preferred_element_type=jnp.float32)
        o_ref[pl.ds(c * _CH, _CH), :] = excl + tot
        tot = tot + excl[_CH - 1 : _CH, :] + xc[_CH - 1 : _CH, :]
    carry_ref[...] = tot


def kernel(x):
    x2 = x.reshape(_B * _N, _L)
    out = pl.pallas_call(
        _scan_body,
        grid=(_B * _N // _BLK,),
        in_specs=[pl.BlockSpec((_BLK, _L), lambda i: (i, 0))],
        out_specs=pl.BlockSpec((_BLK, _L), lambda i: (i, 0)),
        out_shape=jax.ShapeDtypeStruct((_B * _N, _L), jnp.float32),
        scratch_shapes=[pltpu.VMEM((1, _L), jnp.float32)],
    )(x2)
    return out.reshape(_B, _N, _L)
